# baseline (speedup 1.0000x reference)
import functools

import jax
import jax.numpy as jnp
from jax import lax
from jax.experimental import pallas as pl
from jax.experimental.pallas import tpu as pltpu
from jax.experimental.pallas import tpu_sc as plsc

B = 4096
L = 50
LP = 56
WS = 64
FS = 16
NC = 2
NS = 16
NW = NC * NS
BPW = B // NW
RPW = BPW * LP
CB = 2
CH = CB * LP
NCHUNK = BPW // CB
BB = 16


def _sc_gather(Wv, pf1, pf2, xi, li, ri, lei, rei):
    mesh = plsc.VectorSubcoreMesh(core_axis_name="c", subcore_axis_name="s")

    @functools.partial(
        pl.kernel,
        mesh=mesh,
        out_type=(
            jax.ShapeDtypeStruct((B * LP, 128), jnp.float32),
            jax.ShapeDtypeStruct((B, 128), jnp.float32),
        ),
        scratch_types=dict(
            xi_v=pltpu.VMEM((NCHUNK, CH), jnp.int32),
            li_v=pltpu.VMEM((NCHUNK, CH), jnp.int32),
            ri_v=pltpu.VMEM((NCHUNK, CH), jnp.int32),
            le_v=pltpu.VMEM((BPW,), jnp.int32),
            re_v=pltpu.VMEM((BPW,), jnp.int32),
            xrow=pltpu.VMEM((3, CH, WS), jnp.float32),
            ldrow=pltpu.VMEM((3, CH, FS), jnp.float32),
            rdrow=pltpu.VMEM((3, CH, FS), jnp.float32),
            lerow=pltpu.VMEM((BPW, WS), jnp.float32),
            rerow=pltpu.VMEM((BPW, WS), jnp.float32),
            gsem=pltpu.SemaphoreType.DMA((3,)),
            wsem=pltpu.SemaphoreType.DMA((3,)),
            esem=pltpu.SemaphoreType.DMA,
        ),
        compiler_params=pltpu.CompilerParams(use_tc_tiling_on_sc=False),
    )
    def k(Wv_h, pf1_h, pf2_h, xi_h, li_h, ri_h, lei_h, rei_h,
          a_h, e_h,
          xi_v, li_v, ri_v, le_v, re_v,
          xrow, ldrow, rdrow, lerow, rerow, gsem, wsem, esem):
        wid = lax.axis_index("s") * NC + lax.axis_index("c")
        r0 = wid * RPW
        b0 = wid * BPW
        pltpu.sync_copy(xi_h.at[wid], xi_v)
        pltpu.sync_copy(li_h.at[wid], li_v)
        pltpu.sync_copy(ri_h.at[wid], ri_v)
        pltpu.sync_copy(lei_h.at[pl.ds(b0, BPW)], le_v)
        pltpu.sync_copy(rei_h.at[pl.ds(b0, BPW)], re_v)

        pltpu.async_copy(Wv_h.at[le_v], lerow, esem)
        pltpu.async_copy(Wv_h.at[re_v], rerow, esem)
        pltpu.make_async_copy(Wv_h.at[le_v], lerow, esem).wait()
        pltpu.make_async_copy(Wv_h.at[re_v], rerow, esem).wait()
        pltpu.async_copy(lerow, e_h.at[pl.ds(b0, BPW), pl.ds(0, WS)], esem)
        pltpu.async_copy(rerow, e_h.at[pl.ds(b0, BPW), pl.ds(WS, WS)], esem)

        def gather_descs(j, p):
            return (
                pltpu.make_async_copy(
                    Wv_h.at[xi_v.at[j]], xrow.at[p], gsem.at[p]),
                pltpu.make_async_copy(
                    pf1_h.at[li_v.at[j]], ldrow.at[p], gsem.at[p]),
                pltpu.make_async_copy(
                    pf2_h.at[ri_v.at[j]], rdrow.at[p], gsem.at[p]),
            )

        def write_descs(j, p):
            w0 = r0 + j * CH
            return (
                pltpu.make_async_copy(
                    xrow.at[p], a_h.at[pl.ds(w0, CH), pl.ds(0, WS)],
                    wsem.at[p]),
                pltpu.make_async_copy(
                    ldrow.at[p], a_h.at[pl.ds(w0, CH), pl.ds(WS, FS)],
                    wsem.at[p]),
                pltpu.make_async_copy(
                    rdrow.at[p], a_h.at[pl.ds(w0, CH), pl.ds(WS + FS, FS)],
                    wsem.at[p]),
            )

        for d in gather_descs(0, 0):
            d.start()

        @pl.loop(0, NCHUNK)
        def _chunk(j):
            p = lax.rem(j, 3)
            pn = lax.rem(j + 1, 3)

            @pl.when(j + 1 < NCHUNK)
            def _prefetch():
                @pl.when(j >= 2)
                def _drain_old_writes():
                    for d in write_descs(j - 2, pn):
                        d.wait()
                for d in gather_descs(j + 1, pn):
                    d.start()

            for d in gather_descs(j, p):
                d.wait()
            for d in write_descs(j, p):
                d.start()

        for d in write_descs(NCHUNK - 2, (NCHUNK - 2) % 3):
            d.wait()
        for d in write_descs(NCHUNK - 1, (NCHUNK - 1) % 3):
            d.wait()
        pltpu.make_async_copy(lerow, e_h.at[pl.ds(b0, BPW), pl.ds(0, WS)], esem).wait()
        pltpu.make_async_copy(rerow, e_h.at[pl.ds(b0, BPW), pl.ds(WS, WS)], esem).wait()

    return k(Wv, pf1, pf2, xi, li, ri, lei, rei)


def _tc_finish(a3, e):
    def body(a_ref, e_ref, xp_ref, xe_ref):
        a = a_ref[:, :L, :]
        ent = e_ref[...]
        xp_ref[...] = a[:, :, :96]
        e1 = jnp.broadcast_to(ent[:, None, 0:WS], (BB, L, WS))
        e2 = jnp.broadcast_to(ent[:, None, WS:2 * WS], (BB, L, WS))
        xe_ref[...] = jnp.concatenate([a[:, :, 0:WS], e1, e2], axis=-1)

    return pl.pallas_call(
        body,
        out_shape=(
            jax.ShapeDtypeStruct((B, L, 96), jnp.float32),
            jax.ShapeDtypeStruct((B, L, 192), jnp.float32),
        ),
        grid=(B // BB,),
        in_specs=[
            pl.BlockSpec((BB, LP, 128), lambda i: (i, 0, 0)),
            pl.BlockSpec((BB, 128), lambda i: (i, 0)),
        ],
        out_specs=(
            pl.BlockSpec((BB, L, 96), lambda i: (i, 0, 0)),
            pl.BlockSpec((BB, L, 192), lambda i: (i, 0, 0)),
        ),
    )(a3, e)


def _pad_idx(v):
    return jnp.pad(v.astype(jnp.int32).reshape(B, L),
                   ((0, 0), (0, LP - L))).reshape(NW, NCHUNK, CH)


def kernel(Wv, pf1, pf2, x, ldist, rdist, leftEnt, rightEnt):
    xi = _pad_idx(x)
    li = _pad_idx(ldist)
    ri = _pad_idx(rdist)
    lei = leftEnt.astype(jnp.int32).reshape(B)
    rei = rightEnt.astype(jnp.int32).reshape(B)
    a, e = _sc_gather(Wv, pf1, pf2, xi, li, ri, lei, rei)
    xp, xe = _tc_finish(a.reshape(B, LP, 128), e)
    return (xp[:, None], xe)


# speedup vs baseline: 1.9355x; 1.9355x over previous
import functools

import jax
import jax.numpy as jnp
from jax import lax
from jax.experimental import pallas as pl
from jax.experimental.pallas import tpu as pltpu
from jax.experimental.pallas import tpu_sc as plsc

B = 4096
L = 50
LP = 56
WS = 64
FS = 16
NC = 2
NS = 16
NW = NC * NS
BPW = B // NW
RPW = BPW * LP
CB = 2
CH = CB * LP
NCHUNK = BPW // CB
BB = 16


def _sc_gather(Wv, pf1, pf2, xi, li, ri, lei, rei):
    mesh = plsc.VectorSubcoreMesh(core_axis_name="c", subcore_axis_name="s")

    @functools.partial(
        pl.kernel,
        mesh=mesh,
        out_type=(
            jax.ShapeDtypeStruct((B * LP, 128), jnp.float32),
            jax.ShapeDtypeStruct((B, 128), jnp.float32),
        ),
        scratch_types=dict(
            xi_v=pltpu.VMEM((NCHUNK, CH), jnp.int32),
            li_v=pltpu.VMEM((NCHUNK, CH), jnp.int32),
            ri_v=pltpu.VMEM((NCHUNK, CH), jnp.int32),
            le_v=pltpu.VMEM((BPW,), jnp.int32),
            re_v=pltpu.VMEM((BPW,), jnp.int32),
            xrow=pltpu.VMEM((3, CH, WS), jnp.float32),
            ldrow=pltpu.VMEM((3, CH, FS), jnp.float32),
            rdrow=pltpu.VMEM((3, CH, FS), jnp.float32),
            lerow=pltpu.VMEM((BPW, WS), jnp.float32),
            rerow=pltpu.VMEM((BPW, WS), jnp.float32),
            gsem=pltpu.SemaphoreType.DMA((3,)),
            wsem=pltpu.SemaphoreType.DMA((3,)),
            esem=pltpu.SemaphoreType.DMA,
        ),
        compiler_params=pltpu.CompilerParams(use_tc_tiling_on_sc=False),
    )
    def k(Wv_h, pf1_h, pf2_h, xi_h, li_h, ri_h, lei_h, rei_h,
          a_h, e_h,
          xi_v, li_v, ri_v, le_v, re_v,
          xrow, ldrow, rdrow, lerow, rerow, gsem, wsem, esem):
        wid = lax.axis_index("s") * NC + lax.axis_index("c")
        r0 = wid * RPW
        b0 = wid * BPW
        pltpu.sync_copy(xi_h.at[wid], xi_v)
        pltpu.sync_copy(li_h.at[wid], li_v)
        pltpu.sync_copy(ri_h.at[wid], ri_v)
        pltpu.sync_copy(lei_h.at[pl.ds(b0, BPW)], le_v)
        pltpu.sync_copy(rei_h.at[pl.ds(b0, BPW)], re_v)

        pltpu.async_copy(Wv_h.at[le_v], lerow, esem)
        pltpu.async_copy(Wv_h.at[re_v], rerow, esem)
        pltpu.make_async_copy(Wv_h.at[le_v], lerow, esem).wait()
        pltpu.make_async_copy(Wv_h.at[re_v], rerow, esem).wait()
        pltpu.async_copy(lerow, e_h.at[pl.ds(b0, BPW), pl.ds(0, WS)], esem)
        pltpu.async_copy(rerow, e_h.at[pl.ds(b0, BPW), pl.ds(WS, WS)], esem)

        def gather_descs(j, p):
            return (
                pltpu.make_async_copy(
                    Wv_h.at[xi_v.at[j]], xrow.at[p], gsem.at[p]),
                pltpu.make_async_copy(
                    pf1_h.at[li_v.at[j]], ldrow.at[p], gsem.at[p]),
                pltpu.make_async_copy(
                    pf2_h.at[ri_v.at[j]], rdrow.at[p], gsem.at[p]),
            )

        def write_descs(j, p):
            w0 = r0 + j * CH
            return (
                pltpu.make_async_copy(
                    xrow.at[p], a_h.at[pl.ds(w0, CH), pl.ds(0, WS)],
                    wsem.at[p]),
                pltpu.make_async_copy(
                    ldrow.at[p], a_h.at[pl.ds(w0, CH), pl.ds(WS, FS)],
                    wsem.at[p]),
                pltpu.make_async_copy(
                    rdrow.at[p], a_h.at[pl.ds(w0, CH), pl.ds(WS + FS, FS)],
                    wsem.at[p]),
            )

        for d in gather_descs(0, 0):
            d.start()

        @pl.loop(0, NCHUNK)
        def _chunk(j):
            p = lax.rem(j, 3)
            pn = lax.rem(j + 1, 3)

            @pl.when(j + 1 < NCHUNK)
            def _prefetch():
                @pl.when(j >= 2)
                def _drain_old_writes():
                    for d in write_descs(j - 2, pn):
                        d.wait()
                for d in gather_descs(j + 1, pn):
                    d.start()

            for d in gather_descs(j, p):
                d.wait()
            for d in write_descs(j, p):
                d.start()

        for d in write_descs(NCHUNK - 2, (NCHUNK - 2) % 3):
            d.wait()
        for d in write_descs(NCHUNK - 1, (NCHUNK - 1) % 3):
            d.wait()
        pltpu.make_async_copy(lerow, e_h.at[pl.ds(b0, BPW), pl.ds(0, WS)], esem).wait()
        pltpu.make_async_copy(rerow, e_h.at[pl.ds(b0, BPW), pl.ds(WS, WS)], esem).wait()

    return k(Wv, pf1, pf2, xi, li, ri, lei, rei)


def _tc_finish(a3, e):
    def body(a_ref, e_ref, xp_ref, xe_ref):
        a = a_ref[:, :L, :]
        ent = e_ref[...]
        xp_ref[...] = a[:, :, :96]
        e1 = jnp.broadcast_to(ent[:, None, 0:WS], (BB, L, WS))
        e2 = jnp.broadcast_to(ent[:, None, WS:2 * WS], (BB, L, WS))
        xe_ref[...] = jnp.concatenate([a[:, :, 0:WS], e1, e2], axis=-1)

    return pl.pallas_call(
        body,
        out_shape=(
            jax.ShapeDtypeStruct((B, L, 96), jnp.float32),
            jax.ShapeDtypeStruct((B, L, 192), jnp.float32),
        ),
        grid=(B // BB,),
        in_specs=[
            pl.BlockSpec((BB, LP, 128), lambda i: (i, 0, 0)),
            pl.BlockSpec((BB, 128), lambda i: (i, 0)),
        ],
        out_specs=(
            pl.BlockSpec((BB, L, 96), lambda i: (i, 0, 0)),
            pl.BlockSpec((BB, L, 192), lambda i: (i, 0, 0)),
        ),
    )(a3, e)


def _pad_idx(v):
    v = v.astype(jnp.int32).reshape(B, L)
    return jnp.concatenate([v, v[:, :LP - L]], axis=1).reshape(NW, NCHUNK, CH)


def kernel(Wv, pf1, pf2, x, ldist, rdist, leftEnt, rightEnt):
    xi = _pad_idx(x)
    li = _pad_idx(ldist)
    ri = _pad_idx(rdist)
    lei = leftEnt.astype(jnp.int32).reshape(B)
    rei = rightEnt.astype(jnp.int32).reshape(B)
    a, e = _sc_gather(Wv, pf1, pf2, xi, li, ri, lei, rei)
    xp, xe = _tc_finish(a.reshape(B, LP, 128), e)
    return (xp[:, None], xe)


# speedup vs baseline: 2.1693x; 1.1208x over previous
import functools

import jax
import jax.numpy as jnp
from jax import lax
from jax.experimental import pallas as pl
from jax.experimental.pallas import tpu as pltpu
from jax.experimental.pallas import tpu_sc as plsc

B = 4096
L = 50
LP = 56
WS = 64
FS = 16
NC = 2
NS = 16
NW = NC * NS
BPW = B // NW
RPW = BPW * LP
CB = 2
CH = CB * LP
NCHUNK = BPW // CB
BB = 32


def _sc_gather(Wv, pf1, pf2, xi, li, ri, lei, rei):
    mesh = plsc.VectorSubcoreMesh(core_axis_name="c", subcore_axis_name="s")

    @functools.partial(
        pl.kernel,
        mesh=mesh,
        out_type=(
            jax.ShapeDtypeStruct((B * LP, 128), jnp.float32),
            jax.ShapeDtypeStruct((B, 128), jnp.float32),
        ),
        scratch_types=dict(
            xi_v=pltpu.VMEM((NCHUNK, CH), jnp.int32),
            li_v=pltpu.VMEM((NCHUNK, CH), jnp.int32),
            ri_v=pltpu.VMEM((NCHUNK, CH), jnp.int32),
            le_v=pltpu.VMEM((BPW,), jnp.int32),
            re_v=pltpu.VMEM((BPW,), jnp.int32),
            xrow=pltpu.VMEM((3, CH, WS), jnp.float32),
            ldrow=pltpu.VMEM((3, CH, FS), jnp.float32),
            rdrow=pltpu.VMEM((3, CH, FS), jnp.float32),
            lerow=pltpu.VMEM((BPW, WS), jnp.float32),
            rerow=pltpu.VMEM((BPW, WS), jnp.float32),
            gsem=pltpu.SemaphoreType.DMA((3,)),
            wsem=pltpu.SemaphoreType.DMA((3,)),
            esem=pltpu.SemaphoreType.DMA,
        ),
        compiler_params=pltpu.CompilerParams(use_tc_tiling_on_sc=False),
    )
    def k(Wv_h, pf1_h, pf2_h, xi_h, li_h, ri_h, lei_h, rei_h,
          a_h, e_h,
          xi_v, li_v, ri_v, le_v, re_v,
          xrow, ldrow, rdrow, lerow, rerow, gsem, wsem, esem):
        wid = lax.axis_index("s") * NC + lax.axis_index("c")
        r0 = wid * RPW
        b0 = wid * BPW
        pltpu.sync_copy(xi_h.at[wid], xi_v)
        pltpu.sync_copy(li_h.at[wid], li_v)
        pltpu.sync_copy(ri_h.at[wid], ri_v)
        pltpu.sync_copy(lei_h.at[pl.ds(b0, BPW)], le_v)
        pltpu.sync_copy(rei_h.at[pl.ds(b0, BPW)], re_v)

        pltpu.async_copy(Wv_h.at[le_v], lerow, esem)
        pltpu.async_copy(Wv_h.at[re_v], rerow, esem)
        pltpu.make_async_copy(Wv_h.at[le_v], lerow, esem).wait()
        pltpu.make_async_copy(Wv_h.at[re_v], rerow, esem).wait()
        pltpu.async_copy(lerow, e_h.at[pl.ds(b0, BPW), pl.ds(0, WS)], esem)
        pltpu.async_copy(rerow, e_h.at[pl.ds(b0, BPW), pl.ds(WS, WS)], esem)

        def gather_descs(j, p):
            return (
                pltpu.make_async_copy(
                    Wv_h.at[xi_v.at[j]], xrow.at[p], gsem.at[p]),
                pltpu.make_async_copy(
                    pf1_h.at[li_v.at[j]], ldrow.at[p], gsem.at[p]),
                pltpu.make_async_copy(
                    pf2_h.at[ri_v.at[j]], rdrow.at[p], gsem.at[p]),
            )

        def write_descs(j, p):
            w0 = r0 + j * CH
            return (
                pltpu.make_async_copy(
                    xrow.at[p], a_h.at[pl.ds(w0, CH), pl.ds(0, WS)],
                    wsem.at[p]),
                pltpu.make_async_copy(
                    ldrow.at[p], a_h.at[pl.ds(w0, CH), pl.ds(WS, FS)],
                    wsem.at[p]),
                pltpu.make_async_copy(
                    rdrow.at[p], a_h.at[pl.ds(w0, CH), pl.ds(WS + FS, FS)],
                    wsem.at[p]),
            )

        for d in gather_descs(0, 0):
            d.start()

        @pl.loop(0, NCHUNK)
        def _chunk(j):
            p = lax.rem(j, 3)
            pn = lax.rem(j + 1, 3)

            @pl.when(j + 1 < NCHUNK)
            def _prefetch():
                @pl.when(j >= 2)
                def _drain_old_writes():
                    for d in write_descs(j - 2, pn):
                        d.wait()
                for d in gather_descs(j + 1, pn):
                    d.start()

            for d in gather_descs(j, p):
                d.wait()
            for d in write_descs(j, p):
                d.start()

        for d in write_descs(NCHUNK - 2, (NCHUNK - 2) % 3):
            d.wait()
        for d in write_descs(NCHUNK - 1, (NCHUNK - 1) % 3):
            d.wait()
        pltpu.make_async_copy(lerow, e_h.at[pl.ds(b0, BPW), pl.ds(0, WS)], esem).wait()
        pltpu.make_async_copy(rerow, e_h.at[pl.ds(b0, BPW), pl.ds(WS, WS)], esem).wait()

    return k(Wv, pf1, pf2, xi, li, ri, lei, rei)


def _tc_finish(a3, e):
    def body(a_ref, e_ref, xp_ref, xe_ref):
        a = a_ref[:, :L, :]
        ent = e_ref[...]
        xp_ref[...] = a[:, :, :96]
        e1 = jnp.broadcast_to(ent[:, None, 0:WS], (BB, L, WS))
        e2 = jnp.broadcast_to(ent[:, None, WS:2 * WS], (BB, L, WS))
        xe_ref[...] = jnp.concatenate([a[:, :, 0:WS], e1, e2], axis=-1)

    return pl.pallas_call(
        body,
        out_shape=(
            jax.ShapeDtypeStruct((B, L, 96), jnp.float32),
            jax.ShapeDtypeStruct((B, L, 192), jnp.float32),
        ),
        grid=(B // BB,),
        in_specs=[
            pl.BlockSpec((BB, LP, 128), lambda i: (i, 0, 0)),
            pl.BlockSpec((BB, 128), lambda i: (i, 0)),
        ],
        out_specs=(
            pl.BlockSpec((BB, L, 96), lambda i: (i, 0, 0)),
            pl.BlockSpec((BB, L, 192), lambda i: (i, 0, 0)),
        ),
    )(a3, e)


def _pad_idx(v):
    v = v.astype(jnp.int32).reshape(B, L)
    return jnp.concatenate([v, v[:, :LP - L]], axis=1).reshape(NW, NCHUNK, CH)


def kernel(Wv, pf1, pf2, x, ldist, rdist, leftEnt, rightEnt):
    xi = _pad_idx(x)
    li = _pad_idx(ldist)
    ri = _pad_idx(rdist)
    lei = leftEnt.astype(jnp.int32).reshape(B)
    rei = rightEnt.astype(jnp.int32).reshape(B)
    a, e = _sc_gather(Wv, pf1, pf2, xi, li, ri, lei, rei)
    xp, xe = _tc_finish(a.reshape(B, LP, 128), e)
    return (xp[:, None], xe)


# speedup vs baseline: 2.2367x; 1.0311x over previous
import functools

import jax
import jax.numpy as jnp
from jax import lax
from jax.experimental import pallas as pl
from jax.experimental.pallas import tpu as pltpu
from jax.experimental.pallas import tpu_sc as plsc

B = 4096
L = 50
LP = 56
WS = 64
FS = 16
NC = 2
NS = 16
NW = NC * NS
BPW = B // NW
RPW = BPW * LP
CB = 2
CH = CB * LP
NCHUNK = BPW // CB
BB = 64


def _sc_gather(Wv, pf1, pf2, xi, li, ri, lei, rei):
    mesh = plsc.VectorSubcoreMesh(core_axis_name="c", subcore_axis_name="s")

    @functools.partial(
        pl.kernel,
        mesh=mesh,
        out_type=(
            jax.ShapeDtypeStruct((B * LP, 128), jnp.float32),
            jax.ShapeDtypeStruct((B, 128), jnp.float32),
        ),
        scratch_types=dict(
            xi_v=pltpu.VMEM((NCHUNK, CH), jnp.int32),
            li_v=pltpu.VMEM((NCHUNK, CH), jnp.int32),
            ri_v=pltpu.VMEM((NCHUNK, CH), jnp.int32),
            le_v=pltpu.VMEM((BPW,), jnp.int32),
            re_v=pltpu.VMEM((BPW,), jnp.int32),
            xrow=pltpu.VMEM((3, CH, WS), jnp.float32),
            ldrow=pltpu.VMEM((3, CH, FS), jnp.float32),
            rdrow=pltpu.VMEM((3, CH, FS), jnp.float32),
            lerow=pltpu.VMEM((BPW, WS), jnp.float32),
            rerow=pltpu.VMEM((BPW, WS), jnp.float32),
            gsem=pltpu.SemaphoreType.DMA((3,)),
            wsem=pltpu.SemaphoreType.DMA((3,)),
            esem=pltpu.SemaphoreType.DMA,
        ),
        compiler_params=pltpu.CompilerParams(use_tc_tiling_on_sc=False),
    )
    def k(Wv_h, pf1_h, pf2_h, xi_h, li_h, ri_h, lei_h, rei_h,
          a_h, e_h,
          xi_v, li_v, ri_v, le_v, re_v,
          xrow, ldrow, rdrow, lerow, rerow, gsem, wsem, esem):
        wid = lax.axis_index("s") * NC + lax.axis_index("c")
        r0 = wid * RPW
        b0 = wid * BPW
        pltpu.sync_copy(xi_h.at[wid], xi_v)
        pltpu.sync_copy(li_h.at[wid], li_v)
        pltpu.sync_copy(ri_h.at[wid], ri_v)
        pltpu.sync_copy(lei_h.at[pl.ds(b0, BPW)], le_v)
        pltpu.sync_copy(rei_h.at[pl.ds(b0, BPW)], re_v)

        pltpu.async_copy(Wv_h.at[le_v], lerow, esem)
        pltpu.async_copy(Wv_h.at[re_v], rerow, esem)
        pltpu.make_async_copy(Wv_h.at[le_v], lerow, esem).wait()
        pltpu.make_async_copy(Wv_h.at[re_v], rerow, esem).wait()
        pltpu.async_copy(lerow, e_h.at[pl.ds(b0, BPW), pl.ds(0, WS)], esem)
        pltpu.async_copy(rerow, e_h.at[pl.ds(b0, BPW), pl.ds(WS, WS)], esem)

        def gather_descs(j, p):
            return (
                pltpu.make_async_copy(
                    Wv_h.at[xi_v.at[j]], xrow.at[p], gsem.at[p]),
                pltpu.make_async_copy(
                    pf1_h.at[li_v.at[j]], ldrow.at[p], gsem.at[p]),
                pltpu.make_async_copy(
                    pf2_h.at[ri_v.at[j]], rdrow.at[p], gsem.at[p]),
            )

        def write_descs(j, p):
            w0 = r0 + j * CH
            return (
                pltpu.make_async_copy(
                    xrow.at[p], a_h.at[pl.ds(w0, CH), pl.ds(0, WS)],
                    wsem.at[p]),
                pltpu.make_async_copy(
                    ldrow.at[p], a_h.at[pl.ds(w0, CH), pl.ds(WS, FS)],
                    wsem.at[p]),
                pltpu.make_async_copy(
                    rdrow.at[p], a_h.at[pl.ds(w0, CH), pl.ds(WS + FS, FS)],
                    wsem.at[p]),
            )

        for d in gather_descs(0, 0):
            d.start()

        @pl.loop(0, NCHUNK)
        def _chunk(j):
            p = lax.rem(j, 3)
            pn = lax.rem(j + 1, 3)

            @pl.when(j + 1 < NCHUNK)
            def _prefetch():
                @pl.when(j >= 2)
                def _drain_old_writes():
                    for d in write_descs(j - 2, pn):
                        d.wait()
                for d in gather_descs(j + 1, pn):
                    d.start()

            for d in gather_descs(j, p):
                d.wait()
            for d in write_descs(j, p):
                d.start()

        for d in write_descs(NCHUNK - 2, (NCHUNK - 2) % 3):
            d.wait()
        for d in write_descs(NCHUNK - 1, (NCHUNK - 1) % 3):
            d.wait()
        pltpu.make_async_copy(lerow, e_h.at[pl.ds(b0, BPW), pl.ds(0, WS)], esem).wait()
        pltpu.make_async_copy(rerow, e_h.at[pl.ds(b0, BPW), pl.ds(WS, WS)], esem).wait()

    return k(Wv, pf1, pf2, xi, li, ri, lei, rei)


def _tc_finish(a3, e):
    def body(a_ref, e_ref, xp_ref, xe_ref):
        a = a_ref[:, :L, :]
        ent = e_ref[...]
        xp_ref[...] = a[:, :, :96]
        e1 = jnp.broadcast_to(ent[:, None, 0:WS], (BB, L, WS))
        e2 = jnp.broadcast_to(ent[:, None, WS:2 * WS], (BB, L, WS))
        xe_ref[...] = jnp.concatenate([a[:, :, 0:WS], e1, e2], axis=-1)

    return pl.pallas_call(
        body,
        out_shape=(
            jax.ShapeDtypeStruct((B, L, 96), jnp.float32),
            jax.ShapeDtypeStruct((B, L, 192), jnp.float32),
        ),
        grid=(B // BB,),
        in_specs=[
            pl.BlockSpec((BB, LP, 128), lambda i: (i, 0, 0)),
            pl.BlockSpec((BB, 128), lambda i: (i, 0)),
        ],
        out_specs=(
            pl.BlockSpec((BB, L, 96), lambda i: (i, 0, 0)),
            pl.BlockSpec((BB, L, 192), lambda i: (i, 0, 0)),
        ),
    )(a3, e)


def _pad_idx(v):
    v = v.astype(jnp.int32).reshape(B, L)
    return jnp.concatenate([v, v[:, :LP - L]], axis=1).reshape(NW, NCHUNK, CH)


def kernel(Wv, pf1, pf2, x, ldist, rdist, leftEnt, rightEnt):
    xi = _pad_idx(x)
    li = _pad_idx(ldist)
    ri = _pad_idx(rdist)
    lei = leftEnt.astype(jnp.int32).reshape(B)
    rei = rightEnt.astype(jnp.int32).reshape(B)
    a, e = _sc_gather(Wv, pf1, pf2, xi, li, ri, lei, rei)
    xp, xe = _tc_finish(a.reshape(B, LP, 128), e)
    return (xp[:, None], xe)


# speedup vs baseline: 2.2572x; 1.0092x over previous
import functools

import jax
import jax.numpy as jnp
from jax import lax
from jax.experimental import pallas as pl
from jax.experimental.pallas import tpu as pltpu
from jax.experimental.pallas import tpu_sc as plsc

B = 4096
L = 50
LP = 56
WS = 64
FS = 16
NC = 2
NS = 16
NW = NC * NS
BPW = B // NW
RPW = BPW * LP
CB = 2
CH = CB * LP
NCHUNK = BPW // CB
BB = 128


def _sc_gather(Wv, pf1, pf2, xi, li, ri, lei, rei):
    mesh = plsc.VectorSubcoreMesh(core_axis_name="c", subcore_axis_name="s")

    @functools.partial(
        pl.kernel,
        mesh=mesh,
        out_type=(
            jax.ShapeDtypeStruct((B * LP, 128), jnp.float32),
            jax.ShapeDtypeStruct((B, 128), jnp.float32),
        ),
        scratch_types=dict(
            xi_v=pltpu.VMEM((NCHUNK, CH), jnp.int32),
            li_v=pltpu.VMEM((NCHUNK, CH), jnp.int32),
            ri_v=pltpu.VMEM((NCHUNK, CH), jnp.int32),
            le_v=pltpu.VMEM((BPW,), jnp.int32),
            re_v=pltpu.VMEM((BPW,), jnp.int32),
            xrow=pltpu.VMEM((3, CH, WS), jnp.float32),
            ldrow=pltpu.VMEM((3, CH, FS), jnp.float32),
            rdrow=pltpu.VMEM((3, CH, FS), jnp.float32),
            lerow=pltpu.VMEM((BPW, WS), jnp.float32),
            rerow=pltpu.VMEM((BPW, WS), jnp.float32),
            gsem=pltpu.SemaphoreType.DMA((3,)),
            wsem=pltpu.SemaphoreType.DMA((3,)),
            esem=pltpu.SemaphoreType.DMA,
        ),
        compiler_params=pltpu.CompilerParams(use_tc_tiling_on_sc=False),
    )
    def k(Wv_h, pf1_h, pf2_h, xi_h, li_h, ri_h, lei_h, rei_h,
          a_h, e_h,
          xi_v, li_v, ri_v, le_v, re_v,
          xrow, ldrow, rdrow, lerow, rerow, gsem, wsem, esem):
        wid = lax.axis_index("s") * NC + lax.axis_index("c")
        r0 = wid * RPW
        b0 = wid * BPW
        pltpu.sync_copy(xi_h.at[wid], xi_v)
        pltpu.sync_copy(li_h.at[wid], li_v)
        pltpu.sync_copy(ri_h.at[wid], ri_v)
        pltpu.sync_copy(lei_h.at[pl.ds(b0, BPW)], le_v)
        pltpu.sync_copy(rei_h.at[pl.ds(b0, BPW)], re_v)

        pltpu.async_copy(Wv_h.at[le_v], lerow, esem)
        pltpu.async_copy(Wv_h.at[re_v], rerow, esem)
        pltpu.make_async_copy(Wv_h.at[le_v], lerow, esem).wait()
        pltpu.make_async_copy(Wv_h.at[re_v], rerow, esem).wait()
        pltpu.async_copy(lerow, e_h.at[pl.ds(b0, BPW), pl.ds(0, WS)], esem)
        pltpu.async_copy(rerow, e_h.at[pl.ds(b0, BPW), pl.ds(WS, WS)], esem)

        def gather_descs(j, p):
            return (
                pltpu.make_async_copy(
                    Wv_h.at[xi_v.at[j]], xrow.at[p], gsem.at[p]),
                pltpu.make_async_copy(
                    pf1_h.at[li_v.at[j]], ldrow.at[p], gsem.at[p]),
                pltpu.make_async_copy(
                    pf2_h.at[ri_v.at[j]], rdrow.at[p], gsem.at[p]),
            )

        def write_descs(j, p):
            w0 = r0 + j * CH
            return (
                pltpu.make_async_copy(
                    xrow.at[p], a_h.at[pl.ds(w0, CH), pl.ds(0, WS)],
                    wsem.at[p]),
                pltpu.make_async_copy(
                    ldrow.at[p], a_h.at[pl.ds(w0, CH), pl.ds(WS, FS)],
                    wsem.at[p]),
                pltpu.make_async_copy(
                    rdrow.at[p], a_h.at[pl.ds(w0, CH), pl.ds(WS + FS, FS)],
                    wsem.at[p]),
            )

        for d in gather_descs(0, 0):
            d.start()

        @pl.loop(0, NCHUNK)
        def _chunk(j):
            p = lax.rem(j, 3)
            pn = lax.rem(j + 1, 3)

            @pl.when(j + 1 < NCHUNK)
            def _prefetch():
                @pl.when(j >= 2)
                def _drain_old_writes():
                    for d in write_descs(j - 2, pn):
                        d.wait()
                for d in gather_descs(j + 1, pn):
                    d.start()

            for d in gather_descs(j, p):
                d.wait()
            for d in write_descs(j, p):
                d.start()

        for d in write_descs(NCHUNK - 2, (NCHUNK - 2) % 3):
            d.wait()
        for d in write_descs(NCHUNK - 1, (NCHUNK - 1) % 3):
            d.wait()
        pltpu.make_async_copy(lerow, e_h.at[pl.ds(b0, BPW), pl.ds(0, WS)], esem).wait()
        pltpu.make_async_copy(rerow, e_h.at[pl.ds(b0, BPW), pl.ds(WS, WS)], esem).wait()

    return k(Wv, pf1, pf2, xi, li, ri, lei, rei)


def _tc_finish(a3, e):
    def body(a_ref, e_ref, xp_ref, xe_ref):
        a = a_ref[:, :L, :]
        ent = e_ref[...]
        xp_ref[...] = a[:, :, :96]
        e1 = jnp.broadcast_to(ent[:, None, 0:WS], (BB, L, WS))
        e2 = jnp.broadcast_to(ent[:, None, WS:2 * WS], (BB, L, WS))
        xe_ref[...] = jnp.concatenate([a[:, :, 0:WS], e1, e2], axis=-1)

    return pl.pallas_call(
        body,
        out_shape=(
            jax.ShapeDtypeStruct((B, L, 96), jnp.float32),
            jax.ShapeDtypeStruct((B, L, 192), jnp.float32),
        ),
        grid=(B // BB,),
        in_specs=[
            pl.BlockSpec((BB, LP, 128), lambda i: (i, 0, 0)),
            pl.BlockSpec((BB, 128), lambda i: (i, 0)),
        ],
        out_specs=(
            pl.BlockSpec((BB, L, 96), lambda i: (i, 0, 0)),
            pl.BlockSpec((BB, L, 192), lambda i: (i, 0, 0)),
        ),
    )(a3, e)


def _pad_idx(v):
    v = v.astype(jnp.int32).reshape(B, L)
    return jnp.concatenate([v, v[:, :LP - L]], axis=1).reshape(NW, NCHUNK, CH)


def kernel(Wv, pf1, pf2, x, ldist, rdist, leftEnt, rightEnt):
    xi = _pad_idx(x)
    li = _pad_idx(ldist)
    ri = _pad_idx(rdist)
    lei = leftEnt.astype(jnp.int32).reshape(B)
    rei = rightEnt.astype(jnp.int32).reshape(B)
    a, e = _sc_gather(Wv, pf1, pf2, xi, li, ri, lei, rei)
    xp, xe = _tc_finish(a.reshape(B, LP, 128), e)
    return (xp[:, None], xe)
